# baseline (speedup 1.0000x reference)
import jax
import jax.numpy as jnp
import numpy as np
from jax.experimental import pallas as pl
from jax.experimental.pallas import tpu as pltpu

B, D, A = 128, 128, 100000
TA = 4096
NSTEP = (A + TA - 1) // TA
TC = 2048
NEG = -1e9
_TINY = float(np.finfo(np.float32).tiny)

_KS0 = np.uint32(0)
_KS1 = np.uint32(42)
_KS = [_KS0, _KS1, np.uint32(0x1BD11BDA) ^ _KS0 ^ _KS1]


def _gumbel_from_f(x1):
    x0 = jnp.zeros_like(x1) + _KS0
    rots = [[13, 15, 26, 6], [17, 29, 16, 24]]
    for i in range(5):
        for r in rots[i % 2]:
            x0 = x0 + x1
            x1 = (x1 << np.uint32(r)) | (x1 >> np.uint32(32 - r))
            x1 = x1 ^ x0
        x0 = x0 + _KS[(i + 1) % 3]
        x1 = x1 + _KS[(i + 2) % 3] + np.uint32(i + 1)
    bits = x0 ^ x1
    fl = jax.lax.bitcast_convert_type(
        (bits >> np.uint32(9)) | np.uint32(0x3F800000), jnp.float32) - 1.0
    u = jnp.maximum(jnp.float32(_TINY), fl + jnp.float32(_TINY))
    return -jnp.log(-jnp.log(u))


def _kern(obs_ref, mask_ref, w1_ref, b1_ref, w2_ref, b2_ref,
          logit_ref, act_ref, h_ref, lscr, best_val, best_idx):
    j = pl.program_id(0)

    @pl.when(j == 0)
    def _():
        h = jnp.dot(obs_ref[...], w1_ref[...],
                    preferred_element_type=jnp.float32)
        h_ref[...] = jnp.maximum(h + b1_ref[...], 0.0)
        best_val[...] = jnp.full((B, 1), -jnp.inf, jnp.float32)
        best_idx[...] = jnp.zeros((B, 1), jnp.int32)

    @pl.when(j < NSTEP)
    def _():
        slot = jax.lax.rem(j, 2)
        for c in range(TA // TC):
            sl = pl.ds(c * TC, TC)
            logit = jnp.dot(h_ref[...], w2_ref[:, sl],
                            preferred_element_type=jnp.float32)
            logit = logit + b2_ref[:, sl]
            logit = jnp.where(mask_ref[:, sl] != 0, NEG, logit)
            logit_ref[:, sl] = logit
            lscr[slot, :, sl] = logit

    @pl.when(j >= 1)
    def _():
        slot = jax.lax.rem(j - 1, 2)
        row_base = jax.lax.broadcasted_iota(
            jnp.uint32, (B, TC), 0) * np.uint32(A)
        colv = jax.lax.broadcasted_iota(jnp.uint32, (B, TC), 1)
        icol = jax.lax.broadcasted_iota(jnp.int32, (B, TC), 1)
        m_all = jnp.full((B, 1), -jnp.inf, jnp.float32)
        idx_all = jnp.zeros((B, 1), jnp.int32)
        for c in range(TA // TC):
            sl = pl.ds(c * TC, TC)
            off = (j - 1) * TA + c * TC
            g = _gumbel_from_f(row_base + colv
                               + (off.astype(jnp.uint32) + _KS1))
            score = lscr[slot, :, sl] + g
            score = jnp.where(icol + off < A, score, -jnp.inf)
            m = jnp.max(score, axis=1, keepdims=True)
            idx = jnp.min(jnp.where(score == m, icol, A), axis=1,
                          keepdims=True) + off
            better = m > m_all
            m_all = jnp.where(better, m, m_all)
            idx_all = jnp.where(better, idx, idx_all)
        better = m_all > best_val[...]
        best_val[...] = jnp.where(better, m_all, best_val[...])
        best_idx[...] = jnp.where(better, idx_all, best_idx[...])

    @pl.when(j == NSTEP)
    def _():
        act_ref[...] = best_idx[...]


@jax.jit
def _run(obs, mask, W1, b1, W2, b2):
    last = NSTEP - 1
    logit, act = pl.pallas_call(
        _kern,
        grid=(NSTEP + 1,),
        in_specs=[
            pl.BlockSpec((B, D), lambda j: (0, 0)),
            pl.BlockSpec((B, TA), lambda j: (0, jnp.minimum(j, last))),
            pl.BlockSpec((D, D), lambda j: (0, 0)),
            pl.BlockSpec((1, D), lambda j: (0, 0)),
            pl.BlockSpec((D, TA), lambda j: (0, jnp.minimum(j, last))),
            pl.BlockSpec((1, TA), lambda j: (0, jnp.minimum(j, last))),
        ],
        out_specs=[
            pl.BlockSpec((B, TA), lambda j: (0, jnp.minimum(j, last))),
            pl.BlockSpec((B, 1), lambda j: (0, 0)),
        ],
        out_shape=[
            jax.ShapeDtypeStruct((B, A), jnp.float32),
            jax.ShapeDtypeStruct((B, 1), jnp.int32),
        ],
        scratch_shapes=[
            pltpu.VMEM((B, D), jnp.float32),
            pltpu.VMEM((2, B, TA), jnp.float32),
            pltpu.VMEM((B, 1), jnp.float32),
            pltpu.VMEM((B, 1), jnp.int32),
        ],
        compiler_params=pltpu.CompilerParams(
            dimension_semantics=("arbitrary",)),
    )(obs, mask.view(jnp.int8), W1,
      b1.reshape(1, D), W2, b2.reshape(1, A))
    return act[:, 0], logit


def kernel(obs_feat, action_mask, W1, b1, W2, b2):
    return _run(obs_feat, action_mask, W1, b1, W2, b2)


# speedup vs baseline: 1.0474x; 1.0474x over previous
import jax
import jax.numpy as jnp
import numpy as np
from jax.experimental import pallas as pl
from jax.experimental.pallas import tpu as pltpu

B, D, A = 128, 128, 100000
TA = 4096
NT = A // TA
TAIL = A - NT * TA
TC = 2048
TAIL_TC = TAIL
NEG = -1e9
_TINY = float(np.finfo(np.float32).tiny)

_KS0 = np.uint32(0)
_KS1 = np.uint32(42)
_KS = [_KS0, _KS1, np.uint32(0x1BD11BDA) ^ _KS0 ^ _KS1]


def _gumbel_from_f(x1):
    x0 = jnp.zeros_like(x1) + _KS0
    rots = [[13, 15, 26, 6], [17, 29, 16, 24]]
    for i in range(5):
        for r in rots[i % 2]:
            x0 = x0 + x1
            x1 = (x1 << np.uint32(r)) | (x1 >> np.uint32(32 - r))
            x1 = x1 ^ x0
        x0 = x0 + _KS[(i + 1) % 3]
        x1 = x1 + _KS[(i + 2) % 3] + np.uint32(i + 1)
    bits = x0 ^ x1
    fl = jax.lax.bitcast_convert_type(
        (bits >> np.uint32(9)) | np.uint32(0x3F800000), jnp.float32) - 1.0
    u = jnp.maximum(jnp.float32(_TINY), fl + jnp.float32(_TINY))
    return -jnp.log(-jnp.log(u))


def _kern(obs_ref, mask_h, w1_ref, b1_ref, w2_h, b2_h,
          logit_h, act_ref,
          h_ref, f0_ref, w2_buf, mask_buf, b2_buf, logit_buf,
          w2_tl, mask_tl, b2_tl, logit_tl,
          best_val, best_idx,
          sem_w2, sem_mask, sem_b2, sem_out, sem_tl):

    def in_copies(start, slot):
        return (
            pltpu.make_async_copy(
                w2_h.at[:, pl.ds(start, TA)], w2_buf.at[slot],
                sem_w2.at[slot]),
            pltpu.make_async_copy(
                mask_h.at[:, pl.ds(start, TA)], mask_buf.at[slot],
                sem_mask.at[slot]),
            pltpu.make_async_copy(
                b2_h.at[:, pl.ds(start, TA)], b2_buf.at[slot],
                sem_b2.at[slot]),
        )

    def out_copy(start, slot):
        return pltpu.make_async_copy(
            logit_buf.at[slot], logit_h.at[:, pl.ds(start, TA)],
            sem_out.at[slot])

    tail_copies = (
        pltpu.make_async_copy(
            w2_h.at[:, pl.ds(NT * TA, TAIL)], w2_tl, sem_tl.at[0]),
        pltpu.make_async_copy(
            mask_h.at[:, pl.ds(NT * TA, TAIL)], mask_tl, sem_tl.at[1]),
        pltpu.make_async_copy(
            b2_h.at[:, pl.ds(NT * TA, TAIL)], b2_tl, sem_tl.at[2]),
    )
    tail_out = pltpu.make_async_copy(
        logit_tl, logit_h.at[:, pl.ds(NT * TA, TAIL)], sem_tl.at[3])

    for c in in_copies(0, 0):
        c.start()
    for c in in_copies(TA, 1):
        c.start()
    for c in tail_copies:
        c.start()

    h = jnp.dot(obs_ref[...], w1_ref[...],
                preferred_element_type=jnp.float32)
    h_ref[...] = jnp.maximum(h + b1_ref[...], 0.0)
    best_val[...] = jnp.full((B, 1), -jnp.inf, jnp.float32)
    best_idx[...] = jnp.zeros((B, 1), jnp.int32)

    def merge(m, idx):
        better = m > best_val[...]
        best_val[...] = jnp.where(better, m, best_val[...])
        best_idx[...] = jnp.where(better, idx, best_idx[...])

    def chunk_scores(w2_ref_2d, mask_ref_2d, b2_ref_2d, logit_ref_2d,
                     gstart, width):
        row_base = jax.lax.broadcasted_iota(jnp.uint32, (B, width), 0) \
            * np.uint32(A)
        colv = jax.lax.broadcasted_iota(jnp.uint32, (B, width), 1)
        icol = jax.lax.broadcasted_iota(jnp.int32, (B, width), 1)
        m_all, idx_all = None, None
        nchunks = w2_ref_2d.shape[1] // width
        for c in range(nchunks):
            sl = pl.ds(c * width, width)
            logit = jnp.dot(h_ref[...], w2_ref_2d[:, sl],
                            preferred_element_type=jnp.float32)
            logit = logit + b2_ref_2d[:, sl]
            logit = jnp.where(mask_ref_2d[:, sl] != 0, NEG, logit)
            logit_ref_2d[:, sl] = logit
            off = gstart + c * width
            g = _gumbel_from_f(row_base + colv
                               + (off.astype(jnp.uint32) + _KS1
                                  if not isinstance(off, int)
                                  else np.uint32(off + 42)))
            score = logit + g
            m = jnp.max(score, axis=1, keepdims=True)
            idx = jnp.min(jnp.where(score == m, icol, A), axis=1,
                          keepdims=True) + off + c * 0
            if m_all is None:
                m_all, idx_all = m, idx
            else:
                better = m > m_all
                m_all = jnp.where(better, m, m_all)
                idx_all = jnp.where(better, idx, idx_all)
        return m_all, idx_all

    def body(i, _):
        slot = jax.lax.rem(i, 2)
        start = pl.multiple_of(i * TA, TA)
        for c in in_copies(start, slot):
            c.wait()

        @pl.when(i >= 2)
        def _():
            out_copy(pl.multiple_of((i - 2) * TA, TA), slot).wait()

        m, idx = chunk_scores(w2_buf.at[slot], mask_buf.at[slot],
                              b2_buf.at[slot], logit_buf.at[slot],
                              start, TC)
        out_copy(start, slot).start()

        @pl.when(i + 2 < NT)
        def _():
            for c in in_copies(pl.multiple_of((i + 2) * TA, TA), slot):
                c.start()

        merge(m, idx)
        return 0

    jax.lax.fori_loop(0, NT, body, 0)

    for c in tail_copies:
        c.wait()
    m, idx = chunk_scores(w2_tl, mask_tl, b2_tl, logit_tl,
                          NT * TA, TAIL_TC)
    tail_out.start()
    merge(m, idx)

    out_copy((NT - 2) * TA, (NT - 2) % 2).wait()
    out_copy((NT - 1) * TA, (NT - 1) % 2).wait()
    tail_out.wait()
    act_ref[...] = best_idx[...]


@jax.jit
def _run(obs, mask, W1, b1, W2, b2):
    logit, act = pl.pallas_call(
        _kern,
        in_specs=[
            pl.BlockSpec(memory_space=pltpu.VMEM),
            pl.BlockSpec(memory_space=pl.ANY),
            pl.BlockSpec(memory_space=pltpu.VMEM),
            pl.BlockSpec(memory_space=pltpu.VMEM),
            pl.BlockSpec(memory_space=pl.ANY),
            pl.BlockSpec(memory_space=pl.ANY),
        ],
        out_specs=[
            pl.BlockSpec(memory_space=pl.ANY),
            pl.BlockSpec(memory_space=pltpu.VMEM),
        ],
        out_shape=[
            jax.ShapeDtypeStruct((B, A), jnp.float32),
            jax.ShapeDtypeStruct((B, 1), jnp.int32),
        ],
        scratch_shapes=[
            pltpu.VMEM((B, D), jnp.float32),
            pltpu.VMEM((B, TA), jnp.uint32),
            pltpu.VMEM((2, D, TA), jnp.float32),
            pltpu.VMEM((2, B, TA), jnp.int8),
            pltpu.VMEM((2, 1, TA), jnp.float32),
            pltpu.VMEM((2, B, TA), jnp.float32),
            pltpu.VMEM((D, TAIL), jnp.float32),
            pltpu.VMEM((B, TAIL), jnp.int8),
            pltpu.VMEM((1, TAIL), jnp.float32),
            pltpu.VMEM((B, TAIL), jnp.float32),
            pltpu.VMEM((B, 1), jnp.float32),
            pltpu.VMEM((B, 1), jnp.int32),
            pltpu.SemaphoreType.DMA((2,)),
            pltpu.SemaphoreType.DMA((2,)),
            pltpu.SemaphoreType.DMA((2,)),
            pltpu.SemaphoreType.DMA((2,)),
            pltpu.SemaphoreType.DMA((4,)),
        ],
    )(obs, mask.view(jnp.int8), W1,
      b1.reshape(1, D), W2, b2.reshape(1, A))
    return act[:, 0], logit


def kernel(obs_feat, action_mask, W1, b1, W2, b2):
    return _run(obs_feat, action_mask, W1, b1, W2, b2)
